# baseline (speedup 1.0000x reference)
import functools

import jax
import jax.numpy as jnp
from jax import lax
from jax.experimental import pallas as pl
from jax.experimental.pallas import tpu as pltpu
from jax.experimental.pallas import tpu_sc as plsc

_TOP_K = 8
_ROWS_PER_BLOCK = 1024


def _matmul_body(x_ref, w_ref, b_ref, logits_ref):
    logits_ref[...] = jnp.dot(
        x_ref[...], w_ref[...],
        preferred_element_type=jnp.float32) + b_ref[...]


def _tc_logits(x, W, b2):
    m, k = x.shape
    n = W.shape[1]
    r = _ROWS_PER_BLOCK
    return pl.pallas_call(
        _matmul_body,
        grid=(m // r,),
        in_specs=[
            pl.BlockSpec((r, k), lambda i: (i, 0)),
            pl.BlockSpec((k, n), lambda i: (0, 0)),
            pl.BlockSpec((1, n), lambda i: (0, 0)),
        ],
        out_specs=pl.BlockSpec((r, n), lambda i: (i, 0)),
        out_shape=jax.ShapeDtypeStruct((m, n), jnp.float32),
        compiler_params=pltpu.CompilerParams(
            dimension_semantics=("arbitrary",),
        ),
    )(x, W, b2)


def _merge_desc(ak, av, bk, bv):
    rbk = lax.rev(bk, (0,))
    rbv = lax.rev(bv, (0,))
    take = ak >= rbk
    hk = jnp.where(take, ak, rbk)
    hv = jnp.where(take, av, rbv)
    return plsc.sort_key_val(hk, hv, descending=True)


def _sc_topk(logits):
    m, n = logits.shape
    info = plsc.get_sparse_core_info()
    nw = info.num_cores * info.num_subcores
    rows = m // nw
    mesh = plsc.VectorSubcoreMesh(core_axis_name="c", subcore_axis_name="s")

    @functools.partial(
        pl.kernel, mesh=mesh,
        out_type=[
            jax.ShapeDtypeStruct((m * _TOP_K,), jnp.float32),
            jax.ShapeDtypeStruct((m * _TOP_K,), jnp.int32),
        ],
        scratch_types=[
            pltpu.VMEM((rows, n), jnp.float32),
            pltpu.VMEM((rows * _TOP_K + 8,), jnp.float32),
            pltpu.VMEM((rows * _TOP_K + 8,), jnp.int32),
        ],
        compiler_params=pltpu.CompilerParams(needs_layout_passes=False),
    )
    def k(logits_hbm, gates_hbm, idx_hbm, lg_v, g_v, i_v):
        wid = lax.axis_index("s") * info.num_cores + lax.axis_index("c")
        base = wid * rows
        pltpu.sync_copy(logits_hbm.at[pl.ds(base, rows), :], lg_v)

        lane = jnp.arange(16, dtype=jnp.int32)
        low8 = lane < _TOP_K

        def body(r, carry):
            ks = []
            vs = []
            for j in range(n // 16):
                vj = lg_v[r, pl.ds(j * 16, 16)]
                ij = lane + (j * 16)
                sk, sv = plsc.sort_key_val(vj, ij, descending=True)
                ks.append(sk)
                vs.append(sv)
            k01, v01 = _merge_desc(ks[0], vs[0], ks[1], vs[1])
            k23, v23 = _merge_desc(ks[2], vs[2], ks[3], vs[3])
            tk, tv = _merge_desc(k01, v01, k23, v23)
            mx = jnp.max(tk)
            e = jnp.exp(tk - mx)
            e = jnp.where(low8, e, jnp.float32(0.0))
            s = jnp.sum(e)
            plsc.store_compressed(g_v.at[pl.ds(r * _TOP_K, 16)], e / s, mask=low8)
            plsc.store_compressed(i_v.at[pl.ds(r * _TOP_K, 16)], tv, mask=low8)
            return carry

        lax.fori_loop(0, rows, body, 0)

        pltpu.sync_copy(g_v.at[pl.ds(0, rows * _TOP_K)],
                        gates_hbm.at[pl.ds(base * _TOP_K, rows * _TOP_K)])
        pltpu.sync_copy(i_v.at[pl.ds(0, rows * _TOP_K)],
                        idx_hbm.at[pl.ds(base * _TOP_K, rows * _TOP_K)])

    return k(logits)


def kernel(x, W, b):
    n = W.shape[1]
    b2 = b.reshape(1, n)
    logits = _tc_logits(x, W, b2)
    gates_flat, idx_flat = _sc_topk(logits)
    m = x.shape[0]
    return (gates_flat.reshape(m, _TOP_K), idx_flat.reshape(m, _TOP_K))


# speedup vs baseline: 1.4603x; 1.4603x over previous
import jax
import jax.numpy as jnp
from jax.experimental import pallas as pl
from jax.experimental.pallas import tpu as pltpu

_TOP_K = 8
_ROWS_PER_BLOCK = 1024


def _make_body(num_tiles):
    def _body(xa_ref, xb_ref, wa_ref, wb_ref, b_ref, gates_ref, idx_ref,
              logits_ref):
        i = pl.program_id(0)

        @pl.when(i > 0)
        def _topk():
            cur = logits_ref[(i - 1) % 3]
            n = cur.shape[-1]
            col = jax.lax.broadcasted_iota(
                jnp.int32, cur.shape, 1).astype(jnp.float32)
            nf = jnp.float32(n)
            neg_inf = jnp.float32(-jnp.inf)
            vals = [jnp.max(cur, axis=-1, keepdims=True)]
            for _ in range(_TOP_K - 1):
                masked = jnp.where(cur >= vals[-1], neg_inf, cur)
                vals.append(jnp.max(masked, axis=-1, keepdims=True))
            idxs = [jnp.min(jnp.where(cur == v, col, nf), axis=-1,
                            keepdims=True) for v in vals]
            top_vals = jnp.concatenate(vals, axis=-1)
            top_idx = jnp.concatenate(idxs, axis=-1)
            e = jnp.exp(top_vals - top_vals[:, :1])
            gates_ref[...] = e / jnp.sum(e, axis=-1, keepdims=True)
            idx_ref[...] = top_idx.astype(jnp.int32)

        @pl.when(i < num_tiles)
        def _matmul():
            logits_ref[i % 3] = (
                jnp.dot(xa_ref[...], wa_ref[...],
                        preferred_element_type=jnp.float32)
                + jnp.dot(xb_ref[...], wb_ref[...],
                          preferred_element_type=jnp.float32)
                + b_ref[...])

    return _body


def kernel(x, W, b):
    m, k = x.shape
    n = W.shape[1]
    r = _ROWS_PER_BLOCK if m % _ROWS_PER_BLOCK == 0 else m
    nt = m // r
    b2 = b.reshape(1, n)
    gates, idx = pl.pallas_call(
        _make_body(nt),
        grid=(nt + 1,),
        in_specs=[
            pl.BlockSpec((r, k // 2), lambda i: (jnp.minimum(i, nt - 1), 0)),
            pl.BlockSpec((r, k // 2), lambda i: (jnp.minimum(i, nt - 1), 1)),
            pl.BlockSpec((k // 2, n), lambda i: (0, 0)),
            pl.BlockSpec((k // 2, n), lambda i: (1, 0)),
            pl.BlockSpec((1, n), lambda i: (0, 0)),
        ],
        out_specs=[
            pl.BlockSpec((r, _TOP_K), lambda i: (jnp.maximum(i, 1) - 1, 0)),
            pl.BlockSpec((r, _TOP_K), lambda i: (jnp.maximum(i, 1) - 1, 0)),
        ],
        out_shape=[
            jax.ShapeDtypeStruct((m, _TOP_K), jnp.float32),
            jax.ShapeDtypeStruct((m, _TOP_K), jnp.int32),
        ],
        scratch_shapes=[pltpu.VMEM((3, r, n), jnp.float32)],
        compiler_params=pltpu.CompilerParams(
            dimension_semantics=("arbitrary",),
        ),
    )(x, x, W, W, b2)
    return gates, idx


# speedup vs baseline: 1.4620x; 1.0012x over previous
import jax
import jax.numpy as jnp
from jax.experimental import pallas as pl
from jax.experimental.pallas import tpu as pltpu

_TOP_K = 8
_ROWS_PER_BLOCK = 1024


def _make_body(num_tiles):
    def _body(x_ref, w_ref, b_ref, gates_ref, idx_ref, logits_ref):
        i = pl.program_id(0)

        @pl.when(i > 0)
        def _topk():
            cur = logits_ref[(i - 1) % 3]
            n = cur.shape[-1]
            col = jax.lax.broadcasted_iota(
                jnp.int32, cur.shape, 1).astype(jnp.float32)
            nf = jnp.float32(n)
            neg_inf = jnp.float32(-jnp.inf)
            vals = [jnp.max(cur, axis=-1, keepdims=True)]
            for _ in range(_TOP_K - 1):
                masked = jnp.where(cur >= vals[-1], neg_inf, cur)
                vals.append(jnp.max(masked, axis=-1, keepdims=True))
            idxs = [jnp.min(jnp.where(cur == v, col, nf), axis=-1,
                            keepdims=True) for v in vals]
            top_vals = jnp.concatenate(vals, axis=-1)
            top_idx = jnp.concatenate(idxs, axis=-1)
            e = jnp.exp(top_vals - top_vals[:, :1])
            gates_ref[...] = e / jnp.sum(e, axis=-1, keepdims=True)
            idx_ref[...] = top_idx.astype(jnp.int32)

        @pl.when(i < num_tiles)
        def _matmul():
            logits_ref[i % 3] = jnp.dot(
                x_ref[...], w_ref[...],
                preferred_element_type=jnp.float32) + b_ref[...]

    return _body


def kernel(x, W, b):
    m, k = x.shape
    n = W.shape[1]
    r = _ROWS_PER_BLOCK if m % _ROWS_PER_BLOCK == 0 else m
    nt = m // r
    b2 = b.reshape(1, n)
    gates, idx = pl.pallas_call(
        _make_body(nt),
        grid=(nt + 1,),
        in_specs=[
            pl.BlockSpec((r, k), lambda i: (jnp.minimum(i, nt - 1), 0)),
            pl.BlockSpec((k, n), lambda i: (0, 0)),
            pl.BlockSpec((1, n), lambda i: (0, 0)),
        ],
        out_specs=[
            pl.BlockSpec((r, _TOP_K), lambda i: (jnp.maximum(i, 1) - 1, 0)),
            pl.BlockSpec((r, _TOP_K), lambda i: (jnp.maximum(i, 1) - 1, 0)),
        ],
        out_shape=[
            jax.ShapeDtypeStruct((m, _TOP_K), jnp.float32),
            jax.ShapeDtypeStruct((m, _TOP_K), jnp.int32),
        ],
        scratch_shapes=[pltpu.VMEM((3, r, n), jnp.float32)],
        compiler_params=pltpu.CompilerParams(
            dimension_semantics=("arbitrary",),
        ),
    )(x, W, b2)
    return gates, idx


# speedup vs baseline: 1.4620x; 1.0000x over previous
import jax
import jax.numpy as jnp
from jax.experimental import pallas as pl
from jax.experimental.pallas import tpu as pltpu

_TOP_K = 8
_ROWS_PER_BLOCK = 1024


def _make_body(num_tiles):
    def _body(x_ref, w_ref, b_ref, gates_ref, idx_ref, logits_ref):
        i = pl.program_id(0)

        @pl.when(i > 0)
        def _topk():
            cur = logits_ref[(i - 1) % 4]
            n = cur.shape[-1]
            col = jax.lax.broadcasted_iota(
                jnp.int32, cur.shape, 1).astype(jnp.float32)
            nf = jnp.float32(n)
            neg_inf = jnp.float32(-jnp.inf)
            vals = [jnp.max(cur, axis=-1, keepdims=True)]
            for _ in range(_TOP_K - 1):
                masked = jnp.where(cur >= vals[-1], neg_inf, cur)
                vals.append(jnp.max(masked, axis=-1, keepdims=True))
            idxs = [jnp.min(jnp.where(cur == v, col, nf), axis=-1,
                            keepdims=True) for v in vals]
            top_vals = jnp.concatenate(vals, axis=-1)
            top_idx = jnp.concatenate(idxs, axis=-1)
            e = jnp.exp(top_vals - top_vals[:, :1])
            gates_ref[...] = e / jnp.sum(e, axis=-1, keepdims=True)
            idx_ref[...] = top_idx.astype(jnp.int32)

        @pl.when(i < num_tiles)
        def _matmul():
            logits_ref[i % 4] = jnp.dot(
                x_ref[...], w_ref[...],
                preferred_element_type=jnp.float32) + b_ref[...]

    return _body


def kernel(x, W, b):
    m, k = x.shape
    n = W.shape[1]
    r = _ROWS_PER_BLOCK if m % _ROWS_PER_BLOCK == 0 else m
    nt = m // r
    b2 = b.reshape(1, n)
    gates, idx = pl.pallas_call(
        _make_body(nt),
        grid=(nt + 1,),
        in_specs=[
            pl.BlockSpec((r, k), lambda i: (jnp.minimum(i, nt - 1), 0)),
            pl.BlockSpec((k, n), lambda i: (0, 0)),
            pl.BlockSpec((1, n), lambda i: (0, 0)),
        ],
        out_specs=[
            pl.BlockSpec((r, _TOP_K), lambda i: (jnp.maximum(i, 1) - 1, 0)),
            pl.BlockSpec((r, _TOP_K), lambda i: (jnp.maximum(i, 1) - 1, 0)),
        ],
        out_shape=[
            jax.ShapeDtypeStruct((m, _TOP_K), jnp.float32),
            jax.ShapeDtypeStruct((m, _TOP_K), jnp.int32),
        ],
        scratch_shapes=[pltpu.VMEM((4, r, n), jnp.float32)],
        compiler_params=pltpu.CompilerParams(
            dimension_semantics=("arbitrary",),
        ),
    )(x, W, b2)
    return gates, idx
